# Initial kernel scaffold; baseline (speedup 1.0000x reference)
import jax
import jax.numpy as jnp
from jax.experimental import pallas as pl


def kernel(x, edge_index, edge_attr, batch, W1, b1, g1, be1, W2, b2, g2, be2, W3, b3, g3, be3, Wc, bc):
    raise NotImplementedError("write your pallas kernel here")


# speedup vs baseline: 5.7759x; 5.7759x over previous
import functools

import jax
import jax.numpy as jnp
from jax import lax
from jax.experimental import pallas as pl
from jax.experimental.pallas import tpu as pltpu
from jax.experimental.pallas import tpu_sc as plsc

_N = 10000
_E = 320000
_FIN = 128
_H = 256
_HH = 128
_C = 40
_G = 64

_NC = 2
_NS = 16
_K = 128
_E_PAD = 323584
_EPW32 = _E_PAD // 32
_EPT = _E_PAD // 16
_NPAD = 10240
_NSTRIPE = _NPAD // 16
_RSTRIPE = _NPAD // 16

_mesh = plsc.VectorSubcoreMesh(core_axis_name="c", subcore_axis_name="s")
_f32 = jnp.float32


def _rsqrt16(d):
    i = lax.bitcast_convert_type(d, jnp.int32)
    i = jnp.int32(0x5F3759DF) - lax.shift_right_arithmetic(i, 1)
    y = lax.bitcast_convert_type(i, _f32)
    for _ in range(3):
        y = y * (1.5 - 0.5 * d * y * y)
    return y


@functools.partial(
    pl.kernel,
    out_type=jax.ShapeDtypeStruct((_NC, _NPAD), _f32),
    mesh=_mesh,
    compiler_params=pltpu.CompilerParams(needs_layout_passes=False),
    scratch_types=[
        pltpu.VMEM((_K,), jnp.int32),
        pltpu.VMEM((_K,), _f32),
        pltpu.VMEM((_NSTRIPE,), _f32),
        pltpu.VMEM_SHARED((_NPAD,), _f32),
    ],
)
def _deg_kernel(col_hbm, ew_hbm, out_hbm, col_v, ew_v, zbuf, acc):
    c = lax.axis_index("c")
    s = lax.axis_index("s")

    @pl.loop(0, _NSTRIPE // 16)
    def _zero(i):
        zbuf[pl.ds(i * 16, 16)] = jnp.zeros((16,), _f32)

    pltpu.sync_copy(zbuf, acc.at[pl.ds(s * _NSTRIPE, _NSTRIPE)])
    plsc.subcore_barrier()

    base0 = (c * _NS + s) * _EPW32

    @pl.loop(0, _EPW32 // _K)
    def _chunk(k):
        b = base0 + k * _K
        pltpu.sync_copy(col_hbm.at[pl.ds(b, _K)], col_v)
        pltpu.sync_copy(ew_hbm.at[pl.ds(b, _K)], ew_v)
        pltpu.sync_copy(ew_v, acc.at[col_v], add=True)

    plsc.subcore_barrier()
    pltpu.sync_copy(acc.at[pl.ds(s * _NSTRIPE, _NSTRIPE)],
                    out_hbm.at[c, pl.ds(s * _NSTRIPE, _NSTRIPE)])


@functools.partial(
    pl.kernel,
    out_type=jax.ShapeDtypeStruct((_E_PAD,), _f32),
    mesh=_mesh,
    compiler_params=pltpu.CompilerParams(needs_layout_passes=False),
    scratch_types=[
        pltpu.VMEM((_NPAD,), _f32),
        pltpu.VMEM((_NPAD,), _f32),
        pltpu.VMEM((_NPAD,), _f32),
        pltpu.VMEM((_K,), jnp.int32),
        pltpu.VMEM((_K,), jnp.int32),
        pltpu.VMEM((_K,), _f32),
        pltpu.VMEM((_K,), _f32),
    ],
)
def _norm_kernel(deg_hbm, row_hbm, col_hbm, ew_hbm, out_hbm,
                 d0, d1, dis, row_v, col_v, ew_v, norm_v):
    c = lax.axis_index("c")
    s = lax.axis_index("s")
    pltpu.sync_copy(deg_hbm.at[0], d0)
    pltpu.sync_copy(deg_hbm.at[1], d1)

    @pl.loop(0, _NPAD // 16)
    def _dis(i):
        sl = pl.ds(i * 16, 16)
        d = d0[sl] + d1[sl] + 1.0
        dis[sl] = _rsqrt16(d)

    base0 = (c * _NS + s) * _EPW32

    @pl.loop(0, _EPW32 // _K)
    def _chunk(k):
        b = base0 + k * _K
        pltpu.sync_copy(row_hbm.at[pl.ds(b, _K)], row_v)
        pltpu.sync_copy(col_hbm.at[pl.ds(b, _K)], col_v)
        pltpu.sync_copy(ew_hbm.at[pl.ds(b, _K)], ew_v)
        for j in range(_K // 16):
            sl = pl.ds(j * 16, 16)
            dr = plsc.load_gather(dis, [row_v[sl]])
            dc = plsc.load_gather(dis, [col_v[sl]])
            norm_v[sl] = dr * ew_v[sl] * dc
        pltpu.sync_copy(norm_v, out_hbm.at[pl.ds(b, _K)])


@functools.partial(
    pl.kernel,
    out_type=(jax.ShapeDtypeStruct((_NPAD, _HH), _f32),
              jax.ShapeDtypeStruct((_NPAD, _HH), _f32)),
    mesh=_mesh,
    compiler_params=pltpu.CompilerParams(needs_layout_passes=False),
    scratch_types=[
        pltpu.VMEM((_K,), jnp.int32),
        pltpu.VMEM((_K,), jnp.int32),
        pltpu.VMEM((_K,), _f32),
        pltpu.VMEM((_K, _HH), _f32),
        pltpu.VMEM_SHARED((_NPAD, _HH), _f32),
        pltpu.SemaphoreType.DMA,
    ],
)
def _spmm_kernel(h0_hbm, h1_hbm, row_hbm, col_hbm, norm_hbm,
                 out0_hbm, out1_hbm, row_v, col_v, norm_v, gbuf, acc, sem):
    c = lax.axis_index("c")
    s = lax.axis_index("s")

    @pl.loop(0, _K)
    def _zero(r):
        for f in range(_HH // 16):
            gbuf[r, pl.ds(f * 16, 16)] = jnp.zeros((16,), _f32)

    for j in range(5):
        pltpu.sync_copy(gbuf,
                        acc.at[pl.ds(s * _RSTRIPE + j * _K, _K)])
    plsc.subcore_barrier()

    base0 = s * _EPT

    @pl.loop(0, _EPT // _K)
    def _chunk(k):
        b = base0 + k * _K
        pltpu.sync_copy(row_hbm.at[pl.ds(b, _K)], row_v)
        pltpu.sync_copy(col_hbm.at[pl.ds(b, _K)], col_v)
        pltpu.sync_copy(norm_hbm.at[pl.ds(b, _K)], norm_v)

        @pl.when(c == 0)
        def _g0():
            pltpu.async_copy(h0_hbm.at[row_v], gbuf, sem).wait()

        @pl.when(c == 1)
        def _g1():
            pltpu.async_copy(h1_hbm.at[row_v], gbuf, sem).wait()

        @pl.loop(0, _K // 16)
        def _scale(j):
            nvv = norm_v[pl.ds(j * 16, 16)]
            for e in range(16):
                nb = lax.broadcast_in_dim(nvv[e], (16,), ())
                r = j * 16 + e
                for f in range(_HH // 16):
                    sl = pl.ds(f * 16, 16)
                    gbuf[r, sl] = gbuf[r, sl] * nb

        pltpu.sync_copy(gbuf, acc.at[col_v], add=True)

    plsc.subcore_barrier()

    @pl.when(c == 0)
    def _w0():
        pltpu.sync_copy(acc.at[pl.ds(s * _RSTRIPE, _RSTRIPE)],
                        out0_hbm.at[pl.ds(s * _RSTRIPE, _RSTRIPE)])

    @pl.when(c == 1)
    def _w1():
        pltpu.sync_copy(acc.at[pl.ds(s * _RSTRIPE, _RSTRIPE)],
                        out1_hbm.at[pl.ds(s * _RSTRIPE, _RSTRIPE)])


def _tc_in_body(x_ref, w_ref, h0_ref, h1_ref):
    h = jnp.dot(x_ref[...], w_ref[...], preferred_element_type=_f32)
    h0_ref[...] = h[:, :_HH]
    h1_ref[...] = h[:, _HH:]


_tc_in = pl.pallas_call(
    _tc_in_body,
    out_shape=[jax.ShapeDtypeStruct((_N, _HH), _f32)] * 2,
)


def _bn_relu(o, g, be):
    mu = jnp.mean(o, axis=0, keepdims=True)
    var = jnp.mean((o - mu) ** 2, axis=0, keepdims=True)
    return jnp.maximum((o - mu) * lax.rsqrt(var + 1e-5) * g + be, 0.0)


def _halves(a0, a1, h0, h1, degp, b, g, be):
    deg = degp[0, :_N] + degp[1, :_N] + 1.0
    sn = (1.0 / deg)[:, None]
    rs = []
    for half, (a, hh) in enumerate(((a0, h0), (a1, h1))):
        lo = half * _HH
        o = a[0:_N, :] + hh[...] * sn + b[0, lo:lo + _HH][None, :]
        rs.append(_bn_relu(o, g[0, lo:lo + _HH][None, :],
                           be[0, lo:lo + _HH][None, :]))
    return rs


def _tc_layer_body(a0, a1, h0, h1, degp, b, g, be, w, o0_ref, o1_ref):
    r0, r1 = _halves(a0, a1, h0, h1, degp, b, g, be)
    hn = (jnp.dot(r0, w[: _HH, :], preferred_element_type=_f32)
          + jnp.dot(r1, w[_HH:, :], preferred_element_type=_f32))
    o0_ref[...] = hn[:, :_HH]
    o1_ref[...] = hn[:, _HH:]


_tc_layer = pl.pallas_call(
    _tc_layer_body,
    out_shape=[jax.ShapeDtypeStruct((_N, _HH), _f32)] * 2,
)


def _tc_final_body(a0, a1, h0, h1, degp, b, g, be, batch_ref, wc, bc, out_ref):
    r0, r1 = _halves(a0, a1, h0, h1, degp, b, g, be)
    bv = batch_ref[0, :]
    iot = lax.broadcasted_iota(jnp.int32, (_G, _N), 0)
    oh = (iot == bv[None, :]).astype(_f32)
    s0 = jnp.dot(oh, r0, preferred_element_type=_f32)
    s1 = jnp.dot(oh, r1, preferred_element_type=_f32)
    cnt = jnp.sum(oh, axis=1)
    inv = (1.0 / jnp.maximum(cnt, 1.0))[:, None]
    logits = (jnp.dot(s0 * inv, wc[: _HH, :], preferred_element_type=_f32)
              + jnp.dot(s1 * inv, wc[_HH:, :], preferred_element_type=_f32)
              + bc[0][None, :])
    m = jnp.max(logits, axis=1, keepdims=True)
    lse = jnp.log(jnp.sum(jnp.exp(logits - m), axis=1, keepdims=True)) + m
    out_ref[...] = logits - lse


_tc_final = pl.pallas_call(
    _tc_final_body,
    out_shape=jax.ShapeDtypeStruct((_G, _C), _f32),
)


def kernel(x, edge_index, edge_attr, batch,
           W1, b1, g1, be1, W2, b2, g2, be2, W3, b3, g3, be3, Wc, bc):
    pad = _E_PAD - _E
    rowp = jnp.concatenate([edge_index[0], jnp.zeros((pad,), jnp.int32)])
    colp = jnp.concatenate([edge_index[1], jnp.zeros((pad,), jnp.int32)])
    ewp = jnp.concatenate([edge_attr.reshape(-1).astype(_f32),
                           jnp.zeros((pad,), _f32)])

    degp = _deg_kernel(colp, ewp)
    normp = _norm_kernel(degp, rowp, colp, ewp)

    b1r, g1r, be1r = b1.reshape(1, -1), g1.reshape(1, -1), be1.reshape(1, -1)
    b2r, g2r, be2r = b2.reshape(1, -1), g2.reshape(1, -1), be2.reshape(1, -1)
    b3r, g3r, be3r = b3.reshape(1, -1), g3.reshape(1, -1), be3.reshape(1, -1)

    h10, h11 = _tc_in(x, W1)
    a10, a11 = _spmm_kernel(h10, h11, rowp, colp, normp)
    h20, h21 = _tc_layer(a10, a11, h10, h11, degp, b1r, g1r, be1r, W2)
    a20, a21 = _spmm_kernel(h20, h21, rowp, colp, normp)
    h30, h31 = _tc_layer(a20, a21, h20, h21, degp, b2r, g2r, be2r, W3)
    a30, a31 = _spmm_kernel(h30, h31, rowp, colp, normp)
    return _tc_final(a30, a31, h30, h31, degp, b3r, g3r, be3r,
                     batch.reshape(1, -1), Wc, bc.reshape(1, -1))


# speedup vs baseline: 7.8886x; 1.3658x over previous
import functools

import jax
import jax.numpy as jnp
from jax import lax
from jax.experimental import pallas as pl
from jax.experimental.pallas import tpu as pltpu
from jax.experimental.pallas import tpu_sc as plsc

_N = 10000
_E = 320000
_FIN = 128
_H = 256
_HH = 128
_C = 40
_G = 64

_NC = 2
_NS = 16
_K = 128
_E_PAD = 327680
_E2 = _E_PAD // _K
_WROWS = _E2 // 32
_TROWS = _E2 // _NS
_NPAD = 10240
_NSTRIPE = _NPAD // 16

_mesh = plsc.VectorSubcoreMesh(core_axis_name="c", subcore_axis_name="s")
_f32 = jnp.float32


def _rsqrt16(d):
    i = lax.bitcast_convert_type(d, jnp.int32)
    i = jnp.int32(0x5F3759DF) - lax.shift_right_arithmetic(i, 1)
    y = lax.bitcast_convert_type(i, _f32)
    for _ in range(3):
        y = y * (1.5 - 0.5 * d * y * y)
    return y


@functools.partial(
    pl.kernel,
    out_type=jax.ShapeDtypeStruct((_NC, _NPAD), _f32),
    mesh=_mesh,
    compiler_params=pltpu.CompilerParams(needs_layout_passes=False),
    scratch_types=[
        pltpu.VMEM((_WROWS, _K), jnp.int32),
        pltpu.VMEM((_WROWS, _K), _f32),
        pltpu.VMEM((_NSTRIPE,), _f32),
        pltpu.VMEM_SHARED((_NPAD,), _f32),
        pltpu.SemaphoreType.DMA,
        pltpu.SemaphoreType.DMA,
    ],
)
def _deg_kernel(col_hbm, ew_hbm, out_hbm, eb_col, eb_ew, zbuf, acc, sem0, sem1):
    c = lax.axis_index("c")
    s = lax.axis_index("s")
    r0 = (c * _NS + s) * _WROWS
    ld0 = pltpu.async_copy(col_hbm.at[pl.ds(r0, _WROWS)], eb_col, sem0)
    ld1 = pltpu.async_copy(ew_hbm.at[pl.ds(r0, _WROWS)], eb_ew, sem0)

    @pl.loop(0, _NSTRIPE // 16)
    def _zero(i):
        zbuf[pl.ds(i * 16, 16)] = jnp.zeros((16,), _f32)

    pltpu.sync_copy(zbuf, acc.at[pl.ds(s * _NSTRIPE, _NSTRIPE)])
    ld0.wait()
    ld1.wait()
    plsc.subcore_barrier()

    @pl.loop(0, _WROWS // 8)
    def _grp(g):
        for j in range(8):
            ch = g * 8 + j
            pltpu.async_copy(eb_ew.at[ch], acc.at[eb_col.at[ch]], sem1,
                             add=True)
        for j in range(8):
            pltpu.make_async_copy(eb_ew.at[0], acc.at[eb_col.at[0]],
                                  sem1).wait()

    plsc.subcore_barrier()
    pltpu.sync_copy(acc.at[pl.ds(s * _NSTRIPE, _NSTRIPE)],
                    out_hbm.at[c, pl.ds(s * _NSTRIPE, _NSTRIPE)])


@functools.partial(
    pl.kernel,
    out_type=jax.ShapeDtypeStruct((_E2, _K), _f32),
    mesh=_mesh,
    compiler_params=pltpu.CompilerParams(needs_layout_passes=False),
    scratch_types=[
        pltpu.VMEM((_NPAD,), _f32),
        pltpu.VMEM((_NPAD,), _f32),
        pltpu.VMEM((_NPAD,), _f32),
        pltpu.VMEM((_WROWS, _K), jnp.int32),
        pltpu.VMEM((_WROWS, _K), jnp.int32),
        pltpu.VMEM((_WROWS, _K), _f32),
        pltpu.VMEM((_WROWS, _K), _f32),
        pltpu.SemaphoreType.DMA,
    ],
)
def _norm_kernel(deg_hbm, row_hbm, col_hbm, ew_hbm, out_hbm,
                 d0, d1, dis, eb_row, eb_col, eb_ew, eb_out, sem):
    c = lax.axis_index("c")
    s = lax.axis_index("s")
    r0 = (c * _NS + s) * _WROWS
    ld0 = pltpu.async_copy(row_hbm.at[pl.ds(r0, _WROWS)], eb_row, sem)
    ld1 = pltpu.async_copy(col_hbm.at[pl.ds(r0, _WROWS)], eb_col, sem)
    ld2 = pltpu.async_copy(ew_hbm.at[pl.ds(r0, _WROWS)], eb_ew, sem)
    pltpu.sync_copy(deg_hbm.at[0], d0)
    pltpu.sync_copy(deg_hbm.at[1], d1)

    @pl.loop(0, _NPAD // 16)
    def _dis(i):
        sl = pl.ds(i * 16, 16)
        d = d0[sl] + d1[sl] + 1.0
        dis[sl] = _rsqrt16(d)

    ld0.wait()
    ld1.wait()
    ld2.wait()

    @pl.loop(0, _WROWS)
    def _chunk(ch):
        for j in range(_K // 16):
            sl = pl.ds(j * 16, 16)
            dr = plsc.load_gather(dis, [eb_row[ch, sl]])
            dc = plsc.load_gather(dis, [eb_col[ch, sl]])
            eb_out[ch, sl] = dr * eb_ew[ch, sl] * dc

    pltpu.sync_copy(eb_out, out_hbm.at[pl.ds(r0, _WROWS)])


_B = 8
_SB = 2 * _B
_NSB = _TROWS // _SB


@functools.partial(
    pl.kernel,
    out_type=(jax.ShapeDtypeStruct((_NPAD, _HH), _f32),
              jax.ShapeDtypeStruct((_NPAD, _HH), _f32)),
    mesh=_mesh,
    compiler_params=pltpu.CompilerParams(needs_layout_passes=False),
    scratch_types=[
        pltpu.VMEM((_SB, _K), jnp.int32),
        pltpu.VMEM((_SB, _K), jnp.int32),
        pltpu.VMEM((_SB, _K), _f32),
        pltpu.VMEM((_K, _HH), _f32),
        pltpu.VMEM((_K, _HH), _f32),
        pltpu.VMEM_SHARED((_NPAD, _HH), _f32),
        pltpu.SemaphoreType.DMA,
        pltpu.SemaphoreType.DMA,
        pltpu.SemaphoreType.DMA,
        pltpu.SemaphoreType.DMA,
        pltpu.SemaphoreType.DMA,
        pltpu.SemaphoreType.DMA,
    ],
)
def _spmm_kernel(h0_hbm, h1_hbm, row_hbm, col_hbm, norm_hbm,
                 out0_hbm, out1_hbm,
                 ib_row, ib_col, ib_norm,
                 gb0, gb1, acc, gs0, gs1, ss0, ss1, iba, ibs):
    c = lax.axis_index("c")
    s = lax.axis_index("s")
    gb = (gb0, gb1)
    gs = (gs0, gs1)
    ss = (ss0, ss1)
    r0 = s * _TROWS

    @pl.loop(0, _K)
    def _zero(r):
        for f in range(_HH // 16):
            gb0[r, pl.ds(f * 16, 16)] = jnp.zeros((16,), _f32)

    for j in range(5):
        pltpu.sync_copy(gb0, acc.at[pl.ds(s * _NSTRIPE + j * _K, _K)])

    pltpu.sync_copy(row_hbm.at[pl.ds(r0, _SB)], ib_row)
    pltpu.sync_copy(col_hbm.at[pl.ds(r0, _SB)], ib_col)
    pltpu.sync_copy(norm_hbm.at[pl.ds(r0, _SB)], ib_norm)
    plsc.subcore_barrier()

    def start_gather(p, b):
        @pl.when(c == 0)
        def _g0():
            pltpu.async_copy(h0_hbm.at[ib_row.at[p]], gb[b], gs[b])

        @pl.when(c == 1)
        def _g1():
            pltpu.async_copy(h1_hbm.at[ib_row.at[p]], gb[b], gs[b])

    def wait_gather(b):
        pltpu.make_async_copy(h0_hbm.at[ib_row.at[0]], gb[b], gs[b]).wait()

    def wait_scatter(b):
        pltpu.make_async_copy(gb[b], acc.at[ib_col.at[0]], ss[b]).wait()

    def reload_half(lo, rbase, sem):
        pltpu.async_copy(row_hbm.at[pl.ds(rbase, _B)],
                         ib_row.at[pl.ds(lo, _B)], sem)
        pltpu.async_copy(col_hbm.at[pl.ds(rbase, _B)],
                         ib_col.at[pl.ds(lo, _B)], sem)
        pltpu.async_copy(norm_hbm.at[pl.ds(rbase, _B)],
                         ib_norm.at[pl.ds(lo, _B)], sem)

    def wait_half(lo, sem):
        for buf in (ib_row, ib_col, ib_norm):
            pltpu.make_async_copy(row_hbm.at[pl.ds(0, _B)],
                                  buf.at[pl.ds(lo, _B)], sem).wait()

    def scale(p, b):
        @pl.loop(0, _K // 16)
        def _s(g):
            nvv = ib_norm[p, pl.ds(g * 16, 16)]
            for e in range(16):
                nb = lax.broadcast_in_dim(nvv[e], (16,), ())
                r = g * 16 + e
                for f in range(_HH // 16):
                    sl = pl.ds(f * 16, 16)
                    gb[b][r, sl] = gb[b][r, sl] * nb

    start_gather(0, 0)

    @pl.loop(0, _NSB)
    def _super(t):
        c0 = r0 + t * _SB

        @pl.loop(0, _B)
        def _pair(j):
            for pb in (0, 1):
                p = 2 * j + pb
                if pb == 1:
                    wait_scatter(0)
                else:
                    @pl.when(jnp.logical_or(t > 0, j > 0))
                    def _a():
                        wait_scatter(1)
                if pb == 1:
                    @pl.when(j == 0)
                    def _rb():
                        reload_half(_B, c0 + _B, ibs)

                    @pl.when(jnp.logical_and(j == 4, t < _NSB - 1))
                    def _ra():
                        reload_half(0, c0 + _SB, iba)

                    @pl.when(j == 3)
                    def _wb():
                        wait_half(_B, ibs)

                    @pl.when(jnp.logical_and(j == 7, t < _NSB - 1))
                    def _wa():
                        wait_half(0, iba)
                pnext = lax.rem(p + 1, _SB)
                skip_last = jnp.logical_and(jnp.logical_and(j == 7, pb == 1),
                                            t == _NSB - 1)

                @pl.when(jnp.logical_not(skip_last))
                def _g():
                    start_gather(pnext, 1 - pb)
                wait_gather(pb)
                scale(p, pb)
                pltpu.async_copy(gb[pb], acc.at[ib_col.at[p]], ss[pb],
                                 add=True)

    wait_scatter(1)
    plsc.subcore_barrier()

    @pl.when(c == 0)
    def _w0():
        pltpu.sync_copy(acc.at[pl.ds(s * _NSTRIPE, _NSTRIPE)],
                        out0_hbm.at[pl.ds(s * _NSTRIPE, _NSTRIPE)])

    @pl.when(c == 1)
    def _w1():
        pltpu.sync_copy(acc.at[pl.ds(s * _NSTRIPE, _NSTRIPE)],
                        out1_hbm.at[pl.ds(s * _NSTRIPE, _NSTRIPE)])


def _tc_in_body(x_ref, w_ref, h0_ref, h1_ref):
    h = jnp.dot(x_ref[...], w_ref[...], preferred_element_type=_f32)
    h0_ref[...] = h[:, :_HH]
    h1_ref[...] = h[:, _HH:]


_tc_in = pl.pallas_call(
    _tc_in_body,
    out_shape=[jax.ShapeDtypeStruct((_N, _HH), _f32)] * 2,
)


def _bn_relu(o, g, be):
    mu = jnp.mean(o, axis=0, keepdims=True)
    var = jnp.mean((o - mu) ** 2, axis=0, keepdims=True)
    return jnp.maximum((o - mu) * lax.rsqrt(var + 1e-5) * g + be, 0.0)


def _halves(a0, a1, h0, h1, degp, b, g, be):
    deg = degp[0, :_N] + degp[1, :_N] + 1.0
    sn = (1.0 / deg)[:, None]
    rs = []
    for half, (a, hh) in enumerate(((a0, h0), (a1, h1))):
        lo = half * _HH
        o = a[0:_N, :] + hh[...] * sn + b[0, lo:lo + _HH][None, :]
        rs.append(_bn_relu(o, g[0, lo:lo + _HH][None, :],
                           be[0, lo:lo + _HH][None, :]))
    return rs


def _tc_layer_body(a0, a1, h0, h1, degp, b, g, be, w, o0_ref, o1_ref):
    r0, r1 = _halves(a0, a1, h0, h1, degp, b, g, be)
    hn = (jnp.dot(r0, w[: _HH, :], preferred_element_type=_f32)
          + jnp.dot(r1, w[_HH:, :], preferred_element_type=_f32))
    o0_ref[...] = hn[:, :_HH]
    o1_ref[...] = hn[:, _HH:]


_tc_layer = pl.pallas_call(
    _tc_layer_body,
    out_shape=[jax.ShapeDtypeStruct((_N, _HH), _f32)] * 2,
)


def _tc_final_body(a0, a1, h0, h1, degp, b, g, be, batch_ref, wc, bc, out_ref):
    r0, r1 = _halves(a0, a1, h0, h1, degp, b, g, be)
    bv = batch_ref[0, :]
    iot = lax.broadcasted_iota(jnp.int32, (_G, _N), 0)
    oh = (iot == bv[None, :]).astype(_f32)
    s0 = jnp.dot(oh, r0, preferred_element_type=_f32)
    s1 = jnp.dot(oh, r1, preferred_element_type=_f32)
    cnt = jnp.sum(oh, axis=1)
    inv = (1.0 / jnp.maximum(cnt, 1.0))[:, None]
    logits = (jnp.dot(s0 * inv, wc[: _HH, :], preferred_element_type=_f32)
              + jnp.dot(s1 * inv, wc[_HH:, :], preferred_element_type=_f32)
              + bc[0][None, :])
    m = jnp.max(logits, axis=1, keepdims=True)
    lse = jnp.log(jnp.sum(jnp.exp(logits - m), axis=1, keepdims=True)) + m
    out_ref[...] = logits - lse


_tc_final = pl.pallas_call(
    _tc_final_body,
    out_shape=jax.ShapeDtypeStruct((_G, _C), _f32),
)


def kernel(x, edge_index, edge_attr, batch,
           W1, b1, g1, be1, W2, b2, g2, be2, W3, b3, g3, be3, Wc, bc):
    pad = _E_PAD - _E
    rowp = jnp.concatenate([edge_index[0],
                            jnp.zeros((pad,), jnp.int32)]).reshape(_E2, _K)
    colp = jnp.concatenate([edge_index[1],
                            jnp.zeros((pad,), jnp.int32)]).reshape(_E2, _K)
    ewp = jnp.concatenate([edge_attr.reshape(-1).astype(_f32),
                           jnp.zeros((pad,), _f32)]).reshape(_E2, _K)

    degp = _deg_kernel(colp, ewp)
    normp = _norm_kernel(degp, rowp, colp, ewp)

    b1r, g1r, be1r = b1.reshape(1, -1), g1.reshape(1, -1), be1.reshape(1, -1)
    b2r, g2r, be2r = b2.reshape(1, -1), g2.reshape(1, -1), be2.reshape(1, -1)
    b3r, g3r, be3r = b3.reshape(1, -1), g3.reshape(1, -1), be3.reshape(1, -1)

    h10, h11 = _tc_in(x, W1)
    a10, a11 = _spmm_kernel(h10, h11, rowp, colp, normp)
    h20, h21 = _tc_layer(a10, a11, h10, h11, degp, b1r, g1r, be1r, W2)
    a20, a21 = _spmm_kernel(h20, h21, rowp, colp, normp)
    h30, h31 = _tc_layer(a20, a21, h20, h21, degp, b2r, g2r, be2r, W3)
    a30, a31 = _spmm_kernel(h30, h31, rowp, colp, normp)
    return _tc_final(a30, a31, h30, h31, degp, b3r, g3r, be3r,
                     batch.reshape(1, -1), Wc, bc.reshape(1, -1))
